# baseline (speedup 1.0000x reference)
import functools
import jax
import jax.numpy as jnp
from jax import lax
from jax.experimental import pallas as pl
from jax.experimental.pallas import tpu as pltpu
from jax.experimental.pallas import tpu_sc as plsc

_TOTAL = 32768
_B = 16
_D = 256
_NC = 2
_NS = 16
_S = 14336
_DC = _D // _NC
_RPT = _S // _NS
_C = 128
_NCH = _RPT // _C
_L = 16
_DL = _DC // _L
_RU = 16

_TCR = 2048

_mesh = plsc.VectorSubcoreMesh(core_axis_name="c", subcore_axis_name="s")


@functools.partial(
    pl.kernel,
    out_type=jax.ShapeDtypeStruct((_B, _D), jnp.float32),
    mesh=_mesh,
    scratch_types=[
        pltpu.VMEM((_L,), jnp.int32),
        pltpu.VMEM((_C, _DC), jnp.float32),
        pltpu.VMEM((_C, _DC), jnp.float32),
        pltpu.VMEM((_B, _DC), jnp.float32),
        pltpu.VMEM((_B,), jnp.int32),
        pltpu.VMEM_SHARED((_B, _DC), jnp.float32),
        pltpu.SemaphoreType.DMA,
        pltpu.SemaphoreType.DMA,
        pltpu.SemaphoreType.DMA,
    ],
)
def _pool_sc(flat_hbm, cu_hbm, out_hbm, cu_v, buf0, buf1, acc, idx_v, shared,
             sem0, sem1, semc):
    cid = lax.axis_index("c")
    sid = lax.axis_index("s")
    rbase = sid * _RPT
    cbase = cid * _DC

    pltpu.async_copy(
        flat_hbm.at[pl.ds(rbase, _C), pl.ds(cbase, _DC)], buf0, sem0)
    cu_cp = pltpu.async_copy(cu_hbm.at[pl.ds(0, _L)], cu_v, semc)
    pltpu.async_copy(
        flat_hbm.at[pl.ds(rbase + _C, _C), pl.ds(cbase, _DC)], buf1, sem1)

    cu_cp.wait()
    cu_vec = cu_v[pl.ds(0, _L)]
    cu_s = [jnp.int32(0)] + [cu_vec[b] for b in range(1, _B)] + [
        jnp.int32(_TOTAL)]

    def seg_of(pos):
        s = jnp.int32(0)
        for b in range(1, _B):
            s = s + jnp.where(cu_s[b] <= pos, jnp.int32(1), jnp.int32(0))
        return s

    zero = jnp.zeros((_L,), jnp.float32)
    for b in range(_B):
        for j in range(_DL):
            acc[b, pl.ds(j * _L, _L)] = zero

    @pl.when(sid == 0)
    def _():
        pltpu.sync_copy(acc, shared)

    idx_v[pl.ds(0, _L)] = lax.iota(jnp.int32, _L)

    def process(g, buf):
        a = rbase + g * _C
        first = seg_of(a)
        last = seg_of(a + (_C - 1))

        @pl.when(first == last)
        def _():
            def grp(i, carry):
                c = carry
                for rr in range(_RU):
                    r = i * _RU + rr
                    c = tuple(c[j] + buf[r, pl.ds(j * _L, _L)]
                              for j in range(_DL))
                return c

            sums = lax.fori_loop(0, _C // _RU, grp, (zero,) * _DL)
            for j in range(_DL):
                acc[first, pl.ds(j * _L, _L)] += sums[j]

        @pl.when(first != last)
        def _():
            for b in range(_B):
                lo = jnp.clip(cu_s[b] - a, 0, _C)
                hi = jnp.clip(cu_s[b + 1] - a, 0, _C)

                def row_body(r, carry):
                    return tuple(carry[j] + buf[r, pl.ds(j * _L, _L)]
                                 for j in range(_DL))

                sums = lax.fori_loop(lo, hi, row_body, (zero,) * _DL)

                @pl.when(hi > lo)
                def _():
                    for j in range(_DL):
                        acc[b, pl.ds(j * _L, _L)] += sums[j]

    pltpu.make_async_copy(
        flat_hbm.at[pl.ds(0, _C), pl.ds(0, _DC)], buf0, sem0).wait()
    process(0, buf0)
    pltpu.async_copy(
        flat_hbm.at[pl.ds(rbase + 2 * _C, _C), pl.ds(cbase, _DC)], buf0, sem0)

    def outer(gg, carry):
        for k in range(2):
            g = 1 + gg * 2 + k
            buf = buf1 if k == 0 else buf0
            sem = sem1 if k == 0 else sem0
            pltpu.make_async_copy(
                flat_hbm.at[pl.ds(0, _C), pl.ds(0, _DC)], buf, sem).wait()
            process(g, buf)

            @pl.when(g + 2 < _NCH)
            def _():
                pltpu.async_copy(
                    flat_hbm.at[pl.ds(rbase + (g + 2) * _C, _C),
                                pl.ds(cbase, _DC)], buf, sem)
        return carry

    lax.fori_loop(0, (_NCH - 1) // 2, outer, 0)

    plsc.subcore_barrier()
    pltpu.sync_copy(acc, shared.at[idx_v], add=True)
    plsc.subcore_barrier()

    @pl.when(sid == 0)
    def _():
        pltpu.sync_copy(shared, out_hbm.at[:, pl.ds(cbase, _DC)])


def _seg_bounds(cu_ref):
    lows = jnp.stack([cu_ref[b] for b in range(_B)])[:, None]
    highs = jnp.stack([cu_ref[b + 1] for b in range(_B)])[:, None]
    return lows, highs


def _tc_body(cu_ref, x_ref, o_ref):
    i = pl.program_id(0)
    pos = jax.lax.broadcasted_iota(jnp.int32, (_B, _TCR), 1) + (_S + i * _TCR)
    lows, highs = _seg_bounds(cu_ref)
    mask = ((pos >= lows) & (pos < highs)).astype(jnp.float32)
    part = jax.lax.dot(mask, x_ref[...],
                       precision=jax.lax.Precision.HIGHEST,
                       preferred_element_type=jnp.float32)

    @pl.when(i == 0)
    def _():
        o_ref[...] = jnp.zeros_like(o_ref)

    o_ref[...] += part


_tc_part = pl.pallas_call(
    _tc_body,
    grid=((_TOTAL - _S) // _TCR,),
    in_specs=[
        pl.BlockSpec(memory_space=pltpu.SMEM),
        pl.BlockSpec((_TCR, _D), lambda i: (i + _S // _TCR, 0)),
    ],
    out_specs=pl.BlockSpec((_B, _D), lambda i: (0, 0)),
    out_shape=jax.ShapeDtypeStruct((_B, _D), jnp.float32),
)


def _merge_body(cu_ref, a_ref, b_ref, o_ref):
    lows, highs = _seg_bounds(cu_ref)
    cnt = (highs - lows).astype(jnp.float32)
    o_ref[...] = (a_ref[...] + b_ref[...]) / jnp.maximum(cnt, 1.0)


_merge = pl.pallas_call(
    _merge_body,
    in_specs=[
        pl.BlockSpec(memory_space=pltpu.SMEM),
        pl.BlockSpec((_B, _D), lambda: (0, 0)),
        pl.BlockSpec((_B, _D), lambda: (0, 0)),
    ],
    out_shape=jax.ShapeDtypeStruct((_B, _D), jnp.float32),
)


@jax.jit
def kernel(flat, cu_seqlens):
    cu = cu_seqlens.astype(jnp.int32)
    sc_part = _pool_sc(flat, cu)
    tc_part = _tc_part(cu, flat)
    return _merge(cu, sc_part, tc_part)


# speedup vs baseline: 1.1464x; 1.1464x over previous
import functools
import jax
import jax.numpy as jnp
from jax import lax
from jax.experimental import pallas as pl
from jax.experimental.pallas import tpu as pltpu
from jax.experimental.pallas import tpu_sc as plsc

_TOTAL = 32768
_B = 16
_D = 256
_NC = 2
_NS = 16
_S = 8192
_DC = _D // _NC
_RPT = _S // _NS
_C = 128
_NCH = _RPT // _C
_L = 16
_DL = _DC // _L
_RU = 16

_TCR = 2048

_mesh = plsc.VectorSubcoreMesh(core_axis_name="c", subcore_axis_name="s")


@functools.partial(
    pl.kernel,
    out_type=jax.ShapeDtypeStruct((_B, _D), jnp.float32),
    mesh=_mesh,
    scratch_types=[
        pltpu.VMEM((_L,), jnp.int32),
        pltpu.VMEM((_C, _DC), jnp.float32),
        pltpu.VMEM((_C, _DC), jnp.float32),
        pltpu.VMEM((_B, _DC), jnp.float32),
        pltpu.VMEM((_B,), jnp.int32),
        pltpu.VMEM_SHARED((_B, _DC), jnp.float32),
        pltpu.SemaphoreType.DMA,
        pltpu.SemaphoreType.DMA,
    ],
)
def _pool_sc(flat_hbm, cu_hbm, out_hbm, cu_v, buf0, buf1, acc, idx_v, shared,
             sem0, sem1):
    cid = lax.axis_index("c")
    sid = lax.axis_index("s")
    rbase = sid * _RPT
    cbase = cid * _DC

    pltpu.async_copy(
        flat_hbm.at[pl.ds(rbase, _C), pl.ds(cbase, _DC)], buf0, sem0)
    pltpu.async_copy(
        flat_hbm.at[pl.ds(rbase + _C, _C), pl.ds(cbase, _DC)], buf1, sem1)

    pltpu.sync_copy(cu_hbm.at[pl.ds(0, _L)], cu_v)
    cu_vec = cu_v[pl.ds(0, _L)]
    cu_s = [jnp.int32(0)] + [cu_vec[b] for b in range(1, _B)] + [
        jnp.int32(_TOTAL)]

    def seg_of(pos):
        s = jnp.int32(0)
        for b in range(1, _B):
            s = s + jnp.where(cu_s[b] <= pos, jnp.int32(1), jnp.int32(0))
        return s

    zero = jnp.zeros((_L,), jnp.float32)
    for b in range(_B):
        for j in range(_DL):
            acc[b, pl.ds(j * _L, _L)] = zero

    @pl.when(sid == 0)
    def _():
        pltpu.sync_copy(acc, shared)

    idx_v[pl.ds(0, _L)] = lax.iota(jnp.int32, _L)

    def process(g, buf):
        a = rbase + g * _C
        first = seg_of(a)
        last = seg_of(a + (_C - 1))

        @pl.when(first == last)
        def _():
            def grp(i, carry):
                c = carry
                for rr in range(_RU):
                    r = i * _RU + rr
                    c = tuple(c[j] + buf[r, pl.ds(j * _L, _L)]
                              for j in range(_DL))
                return c

            sums = lax.fori_loop(0, _C // _RU, grp, (zero,) * _DL)
            for j in range(_DL):
                acc[first, pl.ds(j * _L, _L)] += sums[j]

        @pl.when(first != last)
        def _():
            for b in range(_B):
                lo = jnp.clip(cu_s[b] - a, 0, _C)
                hi = jnp.clip(cu_s[b + 1] - a, 0, _C)

                def row_body(r, carry):
                    return tuple(carry[j] + buf[r, pl.ds(j * _L, _L)]
                                 for j in range(_DL))

                sums = lax.fori_loop(lo, hi, row_body, (zero,) * _DL)

                @pl.when(hi > lo)
                def _():
                    for j in range(_DL):
                        acc[b, pl.ds(j * _L, _L)] += sums[j]

    def outer(gg, carry):
        for k in range(2):
            g = gg * 2 + k
            buf = buf0 if k == 0 else buf1
            sem = sem0 if k == 0 else sem1
            pltpu.make_async_copy(
                flat_hbm.at[pl.ds(0, _C), pl.ds(0, _DC)], buf, sem).wait()
            process(g, buf)

            @pl.when(g + 2 < _NCH)
            def _():
                pltpu.async_copy(
                    flat_hbm.at[pl.ds(rbase + (g + 2) * _C, _C),
                                pl.ds(cbase, _DC)], buf, sem)
        return carry

    lax.fori_loop(0, _NCH // 2, outer, 0)

    plsc.subcore_barrier()
    pltpu.sync_copy(acc, shared.at[idx_v], add=True)
    plsc.subcore_barrier()

    @pl.when(sid == 0)
    def _():
        pltpu.sync_copy(shared, out_hbm.at[:, pl.ds(cbase, _DC)])


def _seg_bounds(cu_ref):
    lows = jnp.stack([cu_ref[b] for b in range(_B)])[:, None]
    highs = jnp.stack([cu_ref[b + 1] for b in range(_B)])[:, None]
    return lows, highs


def _tc_body(cu_ref, x_ref, o_ref):
    i = pl.program_id(0)
    pos = jax.lax.broadcasted_iota(jnp.int32, (_B, _TCR), 1) + (_S + i * _TCR)
    lows, highs = _seg_bounds(cu_ref)
    mask = ((pos >= lows) & (pos < highs)).astype(jnp.float32)
    part = jax.lax.dot(mask, x_ref[...],
                       preferred_element_type=jnp.float32)

    @pl.when(i == 0)
    def _():
        o_ref[...] = jnp.zeros_like(o_ref)

    o_ref[...] += part


_tc_part = pl.pallas_call(
    _tc_body,
    grid=((_TOTAL - _S) // _TCR,),
    in_specs=[
        pl.BlockSpec(memory_space=pltpu.SMEM),
        pl.BlockSpec((_TCR, _D), lambda i: (i + _S // _TCR, 0)),
    ],
    out_specs=pl.BlockSpec((_B, _D), lambda i: (0, 0)),
    out_shape=jax.ShapeDtypeStruct((_B, _D), jnp.float32),
)


def _merge_body(cu_ref, a_ref, b_ref, o_ref):
    lows, highs = _seg_bounds(cu_ref)
    cnt = (highs - lows).astype(jnp.float32)
    o_ref[...] = (a_ref[...] + b_ref[...]) / jnp.maximum(cnt, 1.0)


_merge = pl.pallas_call(
    _merge_body,
    in_specs=[
        pl.BlockSpec(memory_space=pltpu.SMEM),
        pl.BlockSpec((_B, _D), lambda: (0, 0)),
        pl.BlockSpec((_B, _D), lambda: (0, 0)),
    ],
    out_shape=jax.ShapeDtypeStruct((_B, _D), jnp.float32),
)


@jax.jit
def kernel(flat, cu_seqlens):
    cu = cu_seqlens.astype(jnp.int32)
    sc_part = _pool_sc(flat, cu)
    tc_part = _tc_part(cu, flat)
    return _merge(cu, sc_part, tc_part)
